# Initial kernel scaffold; baseline (speedup 1.0000x reference)
import jax
import jax.numpy as jnp
from jax.experimental import pallas as pl


def kernel(points, b0_conv_w, b0_conv_b, b0_w1, b0_b1, b0_w2, b0_b2, b0_w3, b0_b3, b1_conv_w, b1_conv_b, b1_w1, b1_b1, b1_w2, b1_b2, b1_w3, b1_b3, b2_conv_w, b2_conv_b, b2_w1, b2_b1, b2_w2, b2_b2, b2_w3, b2_b3):
    raise NotImplementedError("write your pallas kernel here")


# speedup vs baseline: 2.0765x; 2.0765x over previous
import jax
import jax.numpy as jnp
from jax.experimental import pallas as pl
from jax.experimental.pallas import tpu as pltpu

_K = 16
_C = 64


def _knn_idx(pos, k, offset=1):
    d2 = jnp.sum(jnp.square(pos[:, :, None, :] - pos[:, None, :, :]), axis=-1)
    _, idx = jax.lax.top_k(-d2, k + offset)
    return idx[:, :, offset:].astype(jnp.int32)


def _gcn_kernel(pts_ref, idx_ref, *refs):
    blk_refs = [refs[i * 8:(i + 1) * 8] for i in range(3)]
    out_ref = refs[24]
    sel_ref = refs[25]

    n = pts_ref.shape[1]
    idx = idx_ref[0]
    iota_c = jax.lax.broadcasted_iota(jnp.int32, (n, n), 1)
    for j in range(_K):
        sel_ref[j * n:(j + 1) * n, :] = jnp.where(
            idx[:, j:j + 1] == iota_c, 1.0, 0.0)

    x0 = pts_ref[0]
    col = 0
    for b in range(3):
        wc, bc, wpt, w21, w3h2, b1, b2, b3 = blk_refs[b]
        d = wc.shape[1]
        if b == 0:
            feat = x0
        else:
            feat = out_ref[0, :, 0:col]
        cur = jnp.dot(feat, wc[...],
                      preferred_element_type=jnp.float32) + bc[...]
        pp = jnp.dot(cur, wpt[...], preferred_element_type=jnp.float32)
        p1 = pp[:, 0:_C] + b1[...]
        q1 = pp[:, _C:2 * _C]
        r2 = pp[:, 2 * _C:3 * _C] + b2[...]
        s3 = pp[:, 3 * _C:4 * _C] + b3[...]

        m1 = m2 = m3 = None
        for j in range(_K):
            g = jnp.dot(sel_ref[j * n:(j + 1) * n, :], q1,
                        preferred_element_type=jnp.float32)
            h1 = jnp.maximum(g + p1, 0.0)
            t = jnp.dot(h1, w21[...], preferred_element_type=jnp.float32)
            h2 = jnp.maximum(t[:, 0:_C] + r2, 0.0)
            h3 = (jnp.dot(h2, w3h2[...], preferred_element_type=jnp.float32)
                  + t[:, _C:2 * _C] + s3)
            m1 = h1 if j == 0 else jnp.maximum(m1, h1)
            m2 = h2 if j == 0 else jnp.maximum(m2, h2)
            m3 = h3 if j == 0 else jnp.maximum(m3, h3)

        if b == 0:
            out_ref[0, :, 0:_C] = cur
            col = _C
        out_ref[0, :, col:col + _C] = m3
        out_ref[0, :, col + _C:col + 2 * _C] = m2
        out_ref[0, :, col + 2 * _C:col + 3 * _C] = m1
        out_ref[0, :, col + 3 * _C:col + 3 * _C + d] = cur
        col = col + 3 * _C + d


def kernel(points,
           b0_conv_w, b0_conv_b, b0_w1, b0_b1, b0_w2, b0_b2, b0_w3, b0_b3,
           b1_conv_w, b1_conv_b, b1_w1, b1_b1, b1_w2, b1_b2, b1_w3, b1_b3,
           b2_conv_w, b2_conv_b, b2_w1, b2_b1, b2_w2, b2_b2, b2_w3, b2_b3):
    blocks = [
        (b0_conv_w, b0_conv_b, b0_w1, b0_b1, b0_w2, b0_b2, b0_w3, b0_b3),
        (b1_conv_w, b1_conv_b, b1_w1, b1_b1, b1_w2, b1_b2, b1_w3, b1_b3),
        (b2_conv_w, b2_conv_b, b2_w1, b2_b1, b2_w2, b2_b2, b2_w3, b2_b3),
    ]
    bsz, n, _ = points.shape
    knn = _knn_idx(points, _K)

    args = [points, knn]
    in_specs = [
        pl.BlockSpec((1, n, 3), lambda b: (b, 0, 0)),
        pl.BlockSpec((1, n, _K), lambda b: (b, 0, 0)),
    ]
    out_dim = 0
    for (cw, cb, w1, b1, w2, b2, w3, b3) in blocks:
        d = cw.shape[0]
        wct = jnp.transpose(cw)
        u1 = jnp.transpose(w1[:, 0:d] - w1[:, 2 * d:3 * d])
        v1 = jnp.transpose(w1[:, d:2 * d] + w1[:, 2 * d:3 * d])
        w2h = jnp.transpose(w2[:, 0:_C])
        w2x = jnp.transpose(w2[:, _C:_C + d])
        w3h2 = jnp.transpose(w3[:, 0:_C])
        w3h1 = jnp.transpose(w3[:, _C:2 * _C])
        w3x = jnp.transpose(w3[:, 2 * _C:2 * _C + d])
        wpt = jnp.concatenate([u1, v1, w2x, w3x], axis=1)
        w21 = jnp.concatenate([w2h, w3h1], axis=1)
        args += [wct, cb.reshape(1, d), wpt, w21, w3h2,
                 b1.reshape(1, _C), b2.reshape(1, _C), b3.reshape(1, _C)]
        cin = wct.shape[0]
        in_specs += [
            pl.BlockSpec((cin, d), lambda b: (0, 0)),
            pl.BlockSpec((1, d), lambda b: (0, 0)),
            pl.BlockSpec((d, 4 * _C), lambda b: (0, 0)),
            pl.BlockSpec((_C, 2 * _C), lambda b: (0, 0)),
            pl.BlockSpec((_C, _C), lambda b: (0, 0)),
            pl.BlockSpec((1, _C), lambda b: (0, 0)),
            pl.BlockSpec((1, _C), lambda b: (0, 0)),
            pl.BlockSpec((1, _C), lambda b: (0, 0)),
        ]
        out_dim += 3 * _C + d
    out_dim += _C

    return pl.pallas_call(
        _gcn_kernel,
        out_shape=jax.ShapeDtypeStruct((bsz, n, out_dim), jnp.float32),
        grid_spec=pltpu.PrefetchScalarGridSpec(
            num_scalar_prefetch=0,
            grid=(bsz,),
            in_specs=in_specs,
            out_specs=pl.BlockSpec((1, n, out_dim), lambda b: (b, 0, 0)),
            scratch_shapes=[pltpu.VMEM((_K * n, n), jnp.float32)],
        ),
        compiler_params=pltpu.CompilerParams(
            dimension_semantics=("parallel",)),
    )(*args)
